# baseline (speedup 1.0000x reference)
import jax
import jax.numpy as jnp
from jax import lax
from jax.experimental import pallas as pl
from jax.experimental.pallas import tpu as pltpu
from jax.experimental.pallas import tpu_sc as plsc

N = 10000
E = 320000
D = 128

NC = 2
NS = 16
NW = NC * NS
EW = E // NW
C = 80
NCH = EW // C
NP = 10240
ST = NP // NS
SZC = ST // C


def _matmul_body(x_ref, as_ref, ad_ref, w_ref, xl_ref, af_ref):
    xl = jnp.dot(x_ref[...], w_ref[...], preferred_element_type=jnp.float32)
    xl_ref[...] = xl
    a2 = jnp.concatenate([as_ref[...], ad_ref[...]], axis=0)
    af_ref[...] = lax.dot_general(
        a2, xl, (((1,), (1,)), ((), ())),
        preferred_element_type=jnp.float32)


def _finalize_body(op_ref, dp_ref, b_ref, o_ref):
    acc = op_ref[0] + op_ref[1]
    den = dp_ref[0] + dp_ref[1]
    y = acc / (den + 1e-16) + b_ref[...]
    o_ref[...] = y * 0.5 * (1.0 + lax.erf(y * 0.7071067811865476))


def _edge_body(xl_hbm, af_hbm, ei_hbm,
               outp_hbm, denp_hbm,
               asrc_v, adst_v, sd_a, sd_b, ee_a, ee_b,
               ssc_a, ssc_b, dsc_a, dsc_b, rows_a, rows_b, out_sh, den_sh,
               gsem0, gsem1, isem0, isem1, dsem0, dsem1, ssem0, ssem1):
    cid = lax.axis_index("c")
    sid = lax.axis_index("s")
    wid = cid * NS + sid
    stripe = sid * ST

    sd = (sd_a, sd_b)
    ee2 = (ee_a, ee_b)
    ssc = (ssc_a, ssc_b)
    dsc = (dsc_a, dsc_b)
    rows_v = (rows_a, rows_b)

    pltpu.sync_copy(af_hbm.at[pl.ds(0, NP)], asrc_v)
    pltpu.sync_copy(af_hbm.at[pl.ds(NP, NP)], adst_v)

    def zrow(i, _):
        for j in range(D // 16):
            rows_a[i, pl.ds(j * 16, 16)] = jnp.zeros((16,), jnp.float32)
        return 0
    lax.fori_loop(0, C, zrow, 0)

    for sub in range(SZC):
        pltpu.sync_copy(rows_a, out_sh.at[pl.ds(stripe + sub * C, C)])
    for sub in range(ST // D):
        pltpu.sync_copy(rows_a.at[0],
                        den_sh.at[pl.ds(stripe + sub * D, D)])

    plsc.subcore_barrier()

    gsem = (gsem0, gsem1)
    isem = (isem0, isem1)
    dsem = (dsem0, dsem1)

    def istart(ch, b):
        off = (wid * NCH + ch) * 2 * C
        pltpu.async_copy(ei_hbm.at[pl.ds(off, 2 * C)], sd[b], isem[b])

    def iwait(ch, b):
        off = (wid * NCH + ch) * 2 * C
        pltpu.make_async_copy(ei_hbm.at[pl.ds(off, 2 * C)], sd[b],
                              isem[b]).wait()

    def ee_compute(b):
        def evec(v, _):
            sl = pl.ds(v * 16, 16)
            sv = sd[b][pl.ds(v * 16, 16)]
            dv = sd[b][pl.ds(C + v * 16, 16)]
            ssc[b][sl] = sv
            dsc[b][sl] = dv
            e = (plsc.load_gather(asrc_v, [sv])
                 + plsc.load_gather(adst_v, [dv]))
            e = jnp.where(e >= 0.0, e, 0.2 * e)
            ee2[b][sl] = jnp.exp(e)
            return 0
        lax.fori_loop(0, C // 16, evec, 0)

    def dfire(b):
        pltpu.async_copy(ee2[b], den_sh.at[dsc[b]], dsem[b], add=True)

    def dwait(b):
        pltpu.make_async_copy(ee2[b], den_sh.at[dsc[b]], dsem[b]).wait()

    def gstart(b):
        pltpu.async_copy(xl_hbm.at[ssc[b]], rows_v[b], gsem[b])

    def gwait(b):
        pltpu.make_async_copy(xl_hbm.at[ssc[b]], rows_v[b], gsem[b]).wait()

    def scale(b):
        @plsc.parallel_loop(0, C, step=1, unroll=4)
        def srow(i):
            av = plsc.load_gather(ee2[b], [jnp.full((16,), i, jnp.int32)])
            for j in range(D // 16):
                sl = pl.ds(j * 16, 16)
                rows_v[b][i, sl] = rows_v[b][i, sl] * av

    ssem = (ssem0, ssem1)

    def sfire(b):
        pltpu.async_copy(rows_v[b], out_sh.at[dsc[b]], ssem[b], add=True)

    def swait(b):
        pltpu.make_async_copy(rows_v[b], out_sh.at[dsc[b]],
                              ssem[b]).wait()

    def finish_and_prep_next(ch, b, do_swait, has_next2):
        if has_next2:
            istart(ch + 2, b)
        gwait(b)
        dwait(b)
        iwait(ch + 1, 1 - b)
        if do_swait:
            swait(1 - b)
        ee_compute(1 - b)
        dfire(1 - b)
        gstart(1 - b)
        scale(b)
        sfire(b)

    istart(0, 0)
    iwait(0, 0)
    ee_compute(0)
    dfire(0)
    gstart(0)
    istart(1, 1)

    finish_and_prep_next(0, 0, False, True)

    def pipe(i, _):
        finish_and_prep_next(i * 2 + 1, 1, True, True)
        finish_and_prep_next(i * 2 + 2, 0, True, True)
        return 0
    lax.fori_loop(0, (NCH - 3) // 2, pipe, 0)

    finish_and_prep_next(NCH - 2, 1, True, False)
    gwait(0)
    dwait(0)
    scale(0)
    sfire(0)
    swait(1)
    swait(0)

    plsc.subcore_barrier()

    pltpu.sync_copy(out_sh.at[pl.ds(stripe, ST)],
                    outp_hbm.at[cid, pl.ds(stripe, ST)])
    pltpu.sync_copy(den_sh.at[pl.ds(stripe, ST)],
                    denp_hbm.at[pl.ds(cid * NP + stripe, ST)])


def kernel(x, edge_index, W, att_src, att_dst, bias):
    nb = 10
    bm = NP // nb
    xl, af = pl.pallas_call(
        _matmul_body,
        grid=(nb,),
        in_specs=[
            pl.BlockSpec((bm, D), lambda i: (i, 0)),
            pl.BlockSpec((1, D), lambda i: (0, 0)),
            pl.BlockSpec((1, D), lambda i: (0, 0)),
            pl.BlockSpec((D, D), lambda i: (0, 0)),
        ],
        out_specs=[
            pl.BlockSpec((bm, D), lambda i: (i, 0)),
            pl.BlockSpec((2, bm), lambda i: (0, i)),
        ],
        out_shape=[
            jax.ShapeDtypeStruct((NP, D), jnp.float32),
            jax.ShapeDtypeStruct((2, NP), jnp.float32),
        ],
    )(x, att_src, att_dst, W)

    eit = edge_index.reshape(2, NW, NCH, C).transpose(1, 2, 0, 3)
    mesh = plsc.VectorSubcoreMesh(core_axis_name="c", subcore_axis_name="s")
    outp, denp = pl.kernel(
        _edge_body,
        out_type=[
            jax.ShapeDtypeStruct((NC, NP, D), jnp.float32),
            jax.ShapeDtypeStruct((NC * NP,), jnp.float32),
        ],
        mesh=mesh,
        compiler_params=pltpu.CompilerParams(needs_layout_passes=False),
        scratch_types=[
            pltpu.VMEM((NP,), jnp.float32),
            pltpu.VMEM((NP,), jnp.float32),
            pltpu.VMEM((2 * C,), jnp.int32),
            pltpu.VMEM((2 * C,), jnp.int32),
            pltpu.VMEM((C,), jnp.float32),
            pltpu.VMEM((C,), jnp.float32),
            pltpu.VMEM((C,), jnp.int32),
            pltpu.VMEM((C,), jnp.int32),
            pltpu.VMEM((C,), jnp.int32),
            pltpu.VMEM((C,), jnp.int32),
            pltpu.VMEM((C, D), jnp.float32),
            pltpu.VMEM((C, D), jnp.float32),
            pltpu.VMEM_SHARED((NP, D), jnp.float32),
            pltpu.VMEM_SHARED((NP,), jnp.float32),
            pltpu.SemaphoreType.DMA,
            pltpu.SemaphoreType.DMA,
            pltpu.SemaphoreType.DMA,
            pltpu.SemaphoreType.DMA,
            pltpu.SemaphoreType.DMA,
            pltpu.SemaphoreType.DMA,
            pltpu.SemaphoreType.DMA,
            pltpu.SemaphoreType.DMA,
        ],
    )(xl, af.reshape(2 * NP), eit.reshape(2 * E))

    bn = 1000
    out = pl.pallas_call(
        _finalize_body,
        grid=(N // bn,),
        in_specs=[
            pl.BlockSpec((NC, bn, D), lambda i: (0, i, 0)),
            pl.BlockSpec((NC, bn, 1), lambda i: (0, i, 0)),
            pl.BlockSpec((1, D), lambda i: (0, 0)),
        ],
        out_specs=pl.BlockSpec((bn, D), lambda i: (i, 0)),
        out_shape=jax.ShapeDtypeStruct((N, D), jnp.float32),
    )(outp, denp.reshape(NC, NP, 1), bias.reshape(1, D))
    return out


# speedup vs baseline: 1.1090x; 1.1090x over previous
import jax
import jax.numpy as jnp
from jax import lax
from jax.experimental import pallas as pl
from jax.experimental.pallas import tpu as pltpu
from jax.experimental.pallas import tpu_sc as plsc

N = 10000
E = 320000
D = 128

NC = 2
NS = 16
NW = NC * NS
EW = E // NW
C = 80
NCH = EW // C
NP = 10240
ST = NP // NS
SZC = ST // C


def _matmul_body(x_ref, as_ref, ad_ref, w_ref, xl_ref, af_ref):
    xl = jnp.dot(x_ref[...], w_ref[...], preferred_element_type=jnp.float32)
    xl_ref[...] = xl
    a2 = jnp.concatenate([as_ref[...], ad_ref[...]], axis=0)
    af_ref[...] = lax.dot_general(
        a2, xl, (((1,), (1,)), ((), ())),
        preferred_element_type=jnp.float32)


def _finalize_body(op_ref, dp_ref, b_ref, o_ref):
    acc = op_ref[0] + op_ref[1]
    den = dp_ref[0] + dp_ref[1]
    y = acc / (den + 1e-16) + b_ref[...]
    o_ref[...] = y * 0.5 * (1.0 + lax.erf(y * 0.7071067811865476))


def _edge_body(xl_hbm, af_hbm, ei_hbm,
               outp_hbm, denp_hbm,
               asrc_v, adst_v, s_a, s_b, d_a, d_b, ee_a, ee_b,
               dsc_a, dsc_b, rows_a, rows_b, out_sh, den_sh,
               gsem0, gsem1, isem0, isem1, dsem0, dsem1, ssem0, ssem1):
    cid = lax.axis_index("c")
    sid = lax.axis_index("s")
    wid = cid * NS + sid
    stripe = sid * ST

    s2 = (s_a, s_b)
    d2 = (d_a, d_b)
    ee2 = (ee_a, ee_b)
    dsc = (dsc_a, dsc_b)
    rows_v = (rows_a, rows_b)

    pltpu.sync_copy(af_hbm.at[pl.ds(0, NP)], asrc_v)
    pltpu.sync_copy(af_hbm.at[pl.ds(NP, NP)], adst_v)

    def zrow(i, _):
        for j in range(D // 16):
            rows_a[i, pl.ds(j * 16, 16)] = jnp.zeros((16,), jnp.float32)
        return 0
    lax.fori_loop(0, C, zrow, 0)

    for sub in range(SZC):
        pltpu.sync_copy(rows_a, out_sh.at[pl.ds(stripe + sub * C, C)])
    for sub in range(ST // D):
        pltpu.sync_copy(rows_a.at[0],
                        den_sh.at[pl.ds(stripe + sub * D, D)])

    plsc.subcore_barrier()

    gsem = (gsem0, gsem1)
    isem = (isem0, isem1)
    dsem = (dsem0, dsem1)

    def istart(ch, b):
        off = wid * EW + ch * C
        pltpu.async_copy(ei_hbm.at[pl.ds(off, C)], s2[b], isem[b])
        pltpu.async_copy(ei_hbm.at[pl.ds(E + off, C)], d2[b], isem[b])

    def iwait(ch, b):
        off = wid * EW + ch * C
        pltpu.make_async_copy(ei_hbm.at[pl.ds(off, C)], s2[b],
                              isem[b]).wait()
        pltpu.make_async_copy(ei_hbm.at[pl.ds(E + off, C)], d2[b],
                              isem[b]).wait()

    def ee_compute(b):
        def evec(v, _):
            sl = pl.ds(v * 16, 16)
            sv = s2[b][sl]
            dv = d2[b][sl]
            dsc[b][sl] = dv
            e = (plsc.load_gather(asrc_v, [sv])
                 + plsc.load_gather(adst_v, [dv]))
            e = jnp.where(e >= 0.0, e, 0.2 * e)
            ee2[b][sl] = jnp.exp(e)
            return 0
        lax.fori_loop(0, C // 16, evec, 0)

    def dfire(b):
        pltpu.async_copy(ee2[b], den_sh.at[d2[b]], dsem[b], add=True)

    def dwait(b):
        pltpu.make_async_copy(ee2[b], den_sh.at[d2[b]], dsem[b]).wait()

    def gstart(b):
        pltpu.async_copy(xl_hbm.at[s2[b]], rows_v[b], gsem[b])

    def gwait(b):
        pltpu.make_async_copy(xl_hbm.at[s2[b]], rows_v[b], gsem[b]).wait()

    def scale(b):
        @plsc.parallel_loop(0, C, step=1, unroll=4)
        def srow(i):
            av = plsc.load_gather(ee2[b], [jnp.full((16,), i, jnp.int32)])
            for j in range(D // 16):
                sl = pl.ds(j * 16, 16)
                rows_v[b][i, sl] = rows_v[b][i, sl] * av

    ssem = (ssem0, ssem1)

    def sfire(b):
        pltpu.async_copy(rows_v[b], out_sh.at[dsc[b]], ssem[b], add=True)

    def swait(b):
        pltpu.make_async_copy(rows_v[b], out_sh.at[dsc[b]],
                              ssem[b]).wait()

    def finish_and_prep_next(ch, b, do_swait, has_next2):
        gwait(b)
        dwait(b)
        iwait(ch + 1, 1 - b)
        if do_swait:
            swait(1 - b)
        ee_compute(1 - b)
        dfire(1 - b)
        gstart(1 - b)
        scale(b)
        sfire(b)
        if has_next2:
            istart(ch + 2, b)

    istart(0, 0)
    iwait(0, 0)
    ee_compute(0)
    dfire(0)
    gstart(0)
    istart(1, 1)

    finish_and_prep_next(0, 0, False, True)

    def pipe(i, _):
        finish_and_prep_next(i * 2 + 1, 1, True, True)
        finish_and_prep_next(i * 2 + 2, 0, True, True)
        return 0
    lax.fori_loop(0, (NCH - 3) // 2, pipe, 0)

    finish_and_prep_next(NCH - 2, 1, True, False)
    gwait(0)
    dwait(0)
    scale(0)
    sfire(0)
    swait(1)
    swait(0)

    plsc.subcore_barrier()

    pltpu.sync_copy(out_sh.at[pl.ds(stripe, ST)],
                    outp_hbm.at[cid, pl.ds(stripe, ST)])
    pltpu.sync_copy(den_sh.at[pl.ds(stripe, ST)],
                    denp_hbm.at[pl.ds(cid * NP + stripe, ST)])


def kernel(x, edge_index, W, att_src, att_dst, bias):
    nb = 10
    bm = NP // nb
    xl, af = pl.pallas_call(
        _matmul_body,
        grid=(nb,),
        in_specs=[
            pl.BlockSpec((bm, D), lambda i: (i, 0)),
            pl.BlockSpec((1, D), lambda i: (0, 0)),
            pl.BlockSpec((1, D), lambda i: (0, 0)),
            pl.BlockSpec((D, D), lambda i: (0, 0)),
        ],
        out_specs=[
            pl.BlockSpec((bm, D), lambda i: (i, 0)),
            pl.BlockSpec((2, bm), lambda i: (0, i)),
        ],
        out_shape=[
            jax.ShapeDtypeStruct((NP, D), jnp.float32),
            jax.ShapeDtypeStruct((2, NP), jnp.float32),
        ],
    )(x, att_src, att_dst, W)

    mesh = plsc.VectorSubcoreMesh(core_axis_name="c", subcore_axis_name="s")
    outp, denp = pl.kernel(
        _edge_body,
        out_type=[
            jax.ShapeDtypeStruct((NC, NP, D), jnp.float32),
            jax.ShapeDtypeStruct((NC * NP,), jnp.float32),
        ],
        mesh=mesh,
        compiler_params=pltpu.CompilerParams(needs_layout_passes=False),
        scratch_types=[
            pltpu.VMEM((NP,), jnp.float32),
            pltpu.VMEM((NP,), jnp.float32),
            pltpu.VMEM((C,), jnp.int32),
            pltpu.VMEM((C,), jnp.int32),
            pltpu.VMEM((C,), jnp.int32),
            pltpu.VMEM((C,), jnp.int32),
            pltpu.VMEM((C,), jnp.float32),
            pltpu.VMEM((C,), jnp.float32),
            pltpu.VMEM((C,), jnp.int32),
            pltpu.VMEM((C,), jnp.int32),
            pltpu.VMEM((C, D), jnp.float32),
            pltpu.VMEM((C, D), jnp.float32),
            pltpu.VMEM_SHARED((NP, D), jnp.float32),
            pltpu.VMEM_SHARED((NP,), jnp.float32),
            pltpu.SemaphoreType.DMA,
            pltpu.SemaphoreType.DMA,
            pltpu.SemaphoreType.DMA,
            pltpu.SemaphoreType.DMA,
            pltpu.SemaphoreType.DMA,
            pltpu.SemaphoreType.DMA,
            pltpu.SemaphoreType.DMA,
            pltpu.SemaphoreType.DMA,
        ],
    )(xl, af.reshape(2 * NP), edge_index.reshape(2 * E))

    bn = 1000
    out = pl.pallas_call(
        _finalize_body,
        grid=(N // bn,),
        in_specs=[
            pl.BlockSpec((NC, bn, D), lambda i: (0, i, 0)),
            pl.BlockSpec((NC, bn, 1), lambda i: (0, i, 0)),
            pl.BlockSpec((1, D), lambda i: (0, 0)),
        ],
        out_specs=pl.BlockSpec((bn, D), lambda i: (i, 0)),
        out_shape=jax.ShapeDtypeStruct((N, D), jnp.float32),
    )(outp, denp.reshape(NC, NP, 1), bias.reshape(1, D))
    return out


# speedup vs baseline: 1.1234x; 1.0130x over previous
import jax
import jax.numpy as jnp
from jax import lax
from jax.experimental import pallas as pl
from jax.experimental.pallas import tpu as pltpu
from jax.experimental.pallas import tpu_sc as plsc

N = 10000
E = 320000
D = 128

NC = 2
NS = 16
NW = NC * NS
EW = E // NW
C = 80
NCH = EW // C
NP = 10240
ST = NP // NS
SZC = ST // C


def _matmul_body(x_ref, as_ref, ad_ref, w_ref, xl_ref, af_ref):
    xl = jnp.dot(x_ref[...], w_ref[...], preferred_element_type=jnp.float32)
    xl_ref[...] = xl
    a2 = jnp.concatenate([as_ref[...], ad_ref[...]], axis=0)
    af_ref[...] = lax.dot_general(
        a2, xl, (((1,), (1,)), ((), ())),
        preferred_element_type=jnp.float32)


def _finalize_body(op_ref, dp_ref, b_ref, o_ref):
    acc = op_ref[0] + op_ref[1]
    den = dp_ref[0] + dp_ref[1]
    y = acc / (den + 1e-16) + b_ref[...]
    o_ref[...] = y * 0.5 * (1.0 + lax.erf(y * 0.7071067811865476))


def _edge_body(xl_hbm, af_hbm, ei_hbm,
               outp_hbm, denp_hbm,
               asrc_v, adst_v, s_a, s_b, d_a, d_b, ee_a, ee_b,
               dsc_a, dsc_b, rows_a, rows_b, out_sh, den_sh,
               gsem0, gsem1, isem0, isem1, dsem0, dsem1, ssem0, ssem1):
    cid = lax.axis_index("c")
    sid = lax.axis_index("s")
    wid = cid * NS + sid
    stripe = sid * ST

    s2 = (s_a, s_b)
    d2 = (d_a, d_b)
    ee2 = (ee_a, ee_b)
    dsc = (dsc_a, dsc_b)
    rows_v = (rows_a, rows_b)

    pltpu.sync_copy(af_hbm.at[pl.ds(0, NP)], asrc_v)
    pltpu.sync_copy(af_hbm.at[pl.ds(NP, NP)], adst_v)

    def zrow(i, _):
        for j in range(D // 16):
            rows_a[i, pl.ds(j * 16, 16)] = jnp.zeros((16,), jnp.float32)
        return 0
    lax.fori_loop(0, C, zrow, 0)

    for sub in range(SZC):
        pltpu.sync_copy(rows_a, out_sh.at[pl.ds(stripe + sub * C, C)])
    for sub in range(ST // D):
        pltpu.sync_copy(rows_a.at[0],
                        den_sh.at[pl.ds(stripe + sub * D, D)])

    plsc.subcore_barrier()

    gsem = (gsem0, gsem1)
    isem = (isem0, isem1)
    dsem = (dsem0, dsem1)

    def istart(ch, b):
        off = wid * EW + ch * C
        pltpu.async_copy(ei_hbm.at[pl.ds(off, C)], s2[b], isem[b])
        pltpu.async_copy(ei_hbm.at[pl.ds(E + off, C)], d2[b], isem[b])

    def iwait(ch, b):
        off = wid * EW + ch * C
        pltpu.make_async_copy(ei_hbm.at[pl.ds(off, C)], s2[b],
                              isem[b]).wait()
        pltpu.make_async_copy(ei_hbm.at[pl.ds(E + off, C)], d2[b],
                              isem[b]).wait()

    def ee_compute(b):
        @plsc.parallel_loop(0, C // 16, step=1, unroll=5)
        def evec(v):
            sl = pl.ds(v * 16, 16)
            sv = s2[b][sl]
            dv = d2[b][sl]
            dsc[b][sl] = dv
            e = (plsc.load_gather(asrc_v, [sv])
                 + plsc.load_gather(adst_v, [dv]))
            e = jnp.where(e >= 0.0, e, 0.2 * e)
            ee2[b][sl] = jnp.exp(e)

    def dfire(b):
        pltpu.async_copy(ee2[b], den_sh.at[d2[b]], dsem[b], add=True)

    def dwait(b):
        pltpu.make_async_copy(ee2[b], den_sh.at[d2[b]], dsem[b]).wait()

    def gstart(b):
        pltpu.async_copy(xl_hbm.at[s2[b]], rows_v[b], gsem[b])

    def gwait(b):
        pltpu.make_async_copy(xl_hbm.at[s2[b]], rows_v[b], gsem[b]).wait()

    def scale(b):
        @plsc.parallel_loop(0, C, step=1, unroll=8)
        def srow(i):
            av = plsc.load_gather(ee2[b], [jnp.full((16,), i, jnp.int32)])
            for j in range(D // 16):
                sl = pl.ds(j * 16, 16)
                rows_v[b][i, sl] = rows_v[b][i, sl] * av

    ssem = (ssem0, ssem1)

    def sfire(b):
        pltpu.async_copy(rows_v[b], out_sh.at[dsc[b]], ssem[b], add=True)

    def swait(b):
        pltpu.make_async_copy(rows_v[b], out_sh.at[dsc[b]],
                              ssem[b]).wait()

    def finish_and_prep_next(ch, b, do_swait, has_next2):
        gwait(b)
        dwait(b)
        iwait(ch + 1, 1 - b)
        if do_swait:
            swait(1 - b)
        ee_compute(1 - b)
        dfire(1 - b)
        gstart(1 - b)
        scale(b)
        sfire(b)
        if has_next2:
            istart(ch + 2, b)

    istart(0, 0)
    iwait(0, 0)
    ee_compute(0)
    dfire(0)
    gstart(0)
    istart(1, 1)

    finish_and_prep_next(0, 0, False, True)

    def pipe(i, _):
        finish_and_prep_next(i * 2 + 1, 1, True, True)
        finish_and_prep_next(i * 2 + 2, 0, True, True)
        return 0
    lax.fori_loop(0, (NCH - 3) // 2, pipe, 0)

    finish_and_prep_next(NCH - 2, 1, True, False)
    gwait(0)
    dwait(0)
    scale(0)
    sfire(0)
    swait(1)
    swait(0)

    plsc.subcore_barrier()

    pltpu.sync_copy(out_sh.at[pl.ds(stripe, ST)],
                    outp_hbm.at[cid, pl.ds(stripe, ST)])
    pltpu.sync_copy(den_sh.at[pl.ds(stripe, ST)],
                    denp_hbm.at[pl.ds(cid * NP + stripe, ST)])


def kernel(x, edge_index, W, att_src, att_dst, bias):
    nb = 10
    bm = NP // nb
    xl, af = pl.pallas_call(
        _matmul_body,
        grid=(nb,),
        in_specs=[
            pl.BlockSpec((bm, D), lambda i: (i, 0)),
            pl.BlockSpec((1, D), lambda i: (0, 0)),
            pl.BlockSpec((1, D), lambda i: (0, 0)),
            pl.BlockSpec((D, D), lambda i: (0, 0)),
        ],
        out_specs=[
            pl.BlockSpec((bm, D), lambda i: (i, 0)),
            pl.BlockSpec((2, bm), lambda i: (0, i)),
        ],
        out_shape=[
            jax.ShapeDtypeStruct((NP, D), jnp.float32),
            jax.ShapeDtypeStruct((2, NP), jnp.float32),
        ],
    )(x, att_src, att_dst, W)

    mesh = plsc.VectorSubcoreMesh(core_axis_name="c", subcore_axis_name="s")
    outp, denp = pl.kernel(
        _edge_body,
        out_type=[
            jax.ShapeDtypeStruct((NC, NP, D), jnp.float32),
            jax.ShapeDtypeStruct((NC * NP,), jnp.float32),
        ],
        mesh=mesh,
        compiler_params=pltpu.CompilerParams(needs_layout_passes=False),
        scratch_types=[
            pltpu.VMEM((NP,), jnp.float32),
            pltpu.VMEM((NP,), jnp.float32),
            pltpu.VMEM((C,), jnp.int32),
            pltpu.VMEM((C,), jnp.int32),
            pltpu.VMEM((C,), jnp.int32),
            pltpu.VMEM((C,), jnp.int32),
            pltpu.VMEM((C,), jnp.float32),
            pltpu.VMEM((C,), jnp.float32),
            pltpu.VMEM((C,), jnp.int32),
            pltpu.VMEM((C,), jnp.int32),
            pltpu.VMEM((C, D), jnp.float32),
            pltpu.VMEM((C, D), jnp.float32),
            pltpu.VMEM_SHARED((NP, D), jnp.float32),
            pltpu.VMEM_SHARED((NP,), jnp.float32),
            pltpu.SemaphoreType.DMA,
            pltpu.SemaphoreType.DMA,
            pltpu.SemaphoreType.DMA,
            pltpu.SemaphoreType.DMA,
            pltpu.SemaphoreType.DMA,
            pltpu.SemaphoreType.DMA,
            pltpu.SemaphoreType.DMA,
            pltpu.SemaphoreType.DMA,
        ],
    )(xl, af.reshape(2 * NP), edge_index.reshape(2 * E))

    bn = 1000
    out = pl.pallas_call(
        _finalize_body,
        grid=(N // bn,),
        in_specs=[
            pl.BlockSpec((NC, bn, D), lambda i: (0, i, 0)),
            pl.BlockSpec((NC, bn, 1), lambda i: (0, i, 0)),
            pl.BlockSpec((1, D), lambda i: (0, 0)),
        ],
        out_specs=pl.BlockSpec((bn, D), lambda i: (i, 0)),
        out_shape=jax.ShapeDtypeStruct((N, D), jnp.float32),
    )(outp, denp.reshape(NC, NP, 1), bias.reshape(1, D))
    return out
